# baseline (speedup 1.0000x reference)
import functools

import jax
import jax.numpy as jnp
from jax.experimental import pallas as pl
from jax.experimental.pallas import tpu as pltpu

_EPS = 1e-5
_GROUPS = 8


def _fused_kernel(x_ref, kt9_ref, a2_ref, gmat_ref, g_ref, b_ref, o_ref,
                  xt_ref, xu9_ref, z_ref, acc_ref, *, d, cin, cout, hw, hw2,
                  groups):
    c2 = 2 * cout

    for i in range(d):
        xt_ref[i * cin:(i + 1) * cin, :] = (
            x_ref[0, :, i * hw:(i + 1) * hw].astype(jnp.bfloat16))

    xu9_ref[...] = jnp.dot(xt_ref[...], kt9_ref[...],
                           preferred_element_type=jnp.float32
                           ).astype(jnp.bfloat16)

    for i in range(d):
        for k in range(9):
            z_ref[k * cin:(k + 1) * cin, i * hw2:(i + 1) * hw2] = \
                xu9_ref[i * cin:(i + 1) * cin, k * hw2:(k + 1) * hw2]

    for i in range(d):
        if i == 0:
            v, ls = 1, 0
        elif i == d - 1:
            v, ls = 2, d - 3
        else:
            v, ls = 0, i - 1
        acc = jnp.zeros((c2, hw2), jnp.float32)
        for t in range(3):
            acc = acc + jnp.dot(
                a2_ref[v, t],
                z_ref[:, (ls + t) * hw2:(ls + t + 1) * hw2],
                preferred_element_type=jnp.float32)
        acc_ref[:, i * hw2:(i + 1) * hw2] = acc

    s1 = jnp.zeros((c2, 1), jnp.float32)
    s2 = jnp.zeros((c2, 1), jnp.float32)
    for i in range(d):
        blk = acc_ref[:, i * hw2:(i + 1) * hw2]
        s1 = s1 + jnp.sum(blk, axis=1, keepdims=True)
        s2 = s2 + jnp.sum(blk * blk, axis=1, keepdims=True)
    cnt = float((cout // groups) * 2 * d * hw2)
    s1g = jnp.dot(gmat_ref[...], s1, preferred_element_type=jnp.float32) / cnt
    s2g = jnp.dot(gmat_ref[...], s2, preferred_element_type=jnp.float32) / cnt
    var = s2g - s1g * s1g
    scale = jax.lax.rsqrt(var + _EPS) * g_ref[...]
    shift = b_ref[...] - s1g * scale

    h2 = o_ref.shape[3]
    w2 = o_ref.shape[4]
    for i in range(d):
        y = acc_ref[:, i * hw2:(i + 1) * hw2] * scale + shift
        y = jnp.maximum(y, 0.0)
        o_ref[0, :, 2 * i] = y[:cout].reshape(cout, h2, w2)
        o_ref[0, :, 2 * i + 1] = y[cout:].reshape(cout, h2, w2)


def _axis_up_matrix(L):
    i = jnp.arange(L)
    m = jnp.zeros((2 * L, L), jnp.float32)
    m = m.at[2 * i, jnp.clip(i - 1, 0, L - 1)].add(0.25)
    m = m.at[2 * i, i].add(0.75)
    m = m.at[2 * i + 1, i].add(0.75)
    m = m.at[2 * i + 1, jnp.clip(i + 1, 0, L - 1)].add(0.25)
    return m


def _depth_fold_coeffs():
    c = [[[[0.0] * 3 for _ in range(3)] for _ in range(2)] for _ in range(3)]
    c[0][0][0] = [0.75, 0.25, 0.0]
    c[0][0][1] = [0.25, 0.75, 0.75]
    c[0][0][2] = [0.0, 0.0, 0.25]
    c[0][1][0] = [0.25, 0.0, 0.0]
    c[0][1][1] = [0.75, 0.75, 0.25]
    c[0][1][2] = [0.0, 0.25, 0.75]
    c[1][0][0] = [0.0, 1.0, 0.75]
    c[1][0][1] = [0.0, 0.0, 0.25]
    c[1][1][0] = [1.0, 0.75, 0.25]
    c[1][1][1] = [0.0, 0.25, 0.75]
    c[2][0][1] = [0.75, 0.25, 0.0]
    c[2][0][2] = [0.25, 0.75, 1.0]
    c[2][1][1] = [0.25, 0.0, 0.0]
    c[2][1][2] = [0.75, 1.0, 0.0]
    return jnp.asarray(c, jnp.float32)


@jax.jit
def _forward(w, gamma, beta, x):
    n, cin, d, h, wd = x.shape
    cout = w.shape[0]
    d2, h2, w2 = 2 * d, 2 * h, 2 * wd
    hw, hw2, r2 = h * wd, h2 * w2, d2 * h2 * w2
    c2 = 2 * cout

    kt = jnp.kron(_axis_up_matrix(h), _axis_up_matrix(wd)).T
    kt3 = kt.reshape(hw, h2, w2)
    ktp = jnp.pad(kt3, ((0, 0), (1, 1), (1, 1)))
    blocks = [ktp[:, 1 + dh:1 + dh + h2, 1 + dw:1 + dw + w2].reshape(hw, hw2)
              for dh in (-1, 0, 1) for dw in (-1, 0, 1)]
    kt9 = jnp.concatenate(blocks, axis=1).astype(jnp.bfloat16)

    coef = _depth_fold_coeffs()
    wp = jnp.transpose(w, (2, 3, 4, 0, 1))
    a = jnp.einsum('vptk,kabyc->vtpyabc', coef, wp)
    a2 = a.reshape(3, 3, c2, 9 * cin).astype(jnp.bfloat16)

    ch = jnp.arange(c2) % cout
    gid = ch // (cout // _GROUPS)
    gmat = (gid[:, None] == gid[None, :]).astype(jnp.float32)

    g2 = jnp.concatenate([gamma, gamma]).reshape(c2, 1).astype(jnp.float32)
    b2 = jnp.concatenate([beta, beta]).reshape(c2, 1).astype(jnp.float32)

    x_r = x.reshape(n, cin, d * hw)

    body = functools.partial(_fused_kernel, d=d, cin=cin, cout=cout,
                             hw=hw, hw2=hw2, groups=_GROUPS)
    out = pl.pallas_call(
        body,
        out_shape=jax.ShapeDtypeStruct((n, cout, d2, h2, w2), jnp.float32),
        grid_spec=pltpu.PrefetchScalarGridSpec(
            num_scalar_prefetch=0,
            grid=(n,),
            in_specs=[
                pl.BlockSpec((1, cin, d * hw), lambda i: (i, 0, 0)),
                pl.BlockSpec((hw, 9 * hw2), lambda i: (0, 0)),
                pl.BlockSpec((3, 3, c2, 9 * cin), lambda i: (0, 0, 0, 0)),
                pl.BlockSpec((c2, c2), lambda i: (0, 0)),
                pl.BlockSpec((c2, 1), lambda i: (0, 0)),
                pl.BlockSpec((c2, 1), lambda i: (0, 0)),
            ],
            out_specs=pl.BlockSpec((1, cout, d2, h2, w2),
                                   lambda i: (i, 0, 0, 0, 0)),
            scratch_shapes=[
                pltpu.VMEM((d * cin, hw), jnp.bfloat16),
                pltpu.VMEM((d * cin, 9 * hw2), jnp.bfloat16),
                pltpu.VMEM((9 * cin, d * hw2), jnp.bfloat16),
                pltpu.VMEM((c2, d * hw2), jnp.float32),
            ],
        ),
        compiler_params=pltpu.CompilerParams(
            dimension_semantics=("parallel",)),
    )(x_r, kt9, a2, gmat, g2, b2)

    return out


def kernel(w, gamma, beta, x):
    return _forward(w, gamma, beta, x)


# speedup vs baseline: 2.4783x; 2.4783x over previous
import functools

import jax
import jax.numpy as jnp
from jax.experimental import pallas as pl
from jax.experimental.pallas import tpu as pltpu

_EPS = 1e-5
_GROUPS = 8


def _fused_kernel(x_ref, kt9_ref, a2_ref, gmat_ref, g_ref, b_ref, o_ref,
                  xt_ref, xu9_ref, z_ref, acc_ref, *, d, cin, cout, hw, hw2,
                  groups):
    c2 = 2 * cout

    for i in range(d):
        xt_ref[i * cin:(i + 1) * cin, :] = (
            x_ref[0, :, i * hw:(i + 1) * hw].astype(jnp.bfloat16))

    xu9_ref[...] = jnp.dot(xt_ref[...], kt9_ref[...],
                           preferred_element_type=jnp.float32
                           ).astype(jnp.bfloat16)

    for i in range(d):
        for k in range(9):
            z_ref[k * cin:(k + 1) * cin, i * hw2:(i + 1) * hw2] = \
                xu9_ref[i * cin:(i + 1) * cin, k * hw2:(k + 1) * hw2]

    for i in range(d):
        if i == 0:
            v, ls = 1, 0
        elif i == d - 1:
            v, ls = 2, d - 3
        else:
            v, ls = 0, i - 1
        acc = jnp.zeros((c2, hw2), jnp.float32)
        for t in range(3):
            acc = acc + jnp.dot(
                a2_ref[v, t],
                z_ref[:, (ls + t) * hw2:(ls + t + 1) * hw2],
                preferred_element_type=jnp.float32)
        acc_ref[:, i * hw2:(i + 1) * hw2] = acc

    s1 = jnp.zeros((c2, 1), jnp.float32)
    s2 = jnp.zeros((c2, 1), jnp.float32)
    for i in range(d):
        blk = acc_ref[:, i * hw2:(i + 1) * hw2]
        s1 = s1 + jnp.sum(blk, axis=1, keepdims=True)
        s2 = s2 + jnp.sum(blk * blk, axis=1, keepdims=True)
    cnt = float((cout // groups) * 2 * d * hw2)
    s1g = jnp.dot(gmat_ref[...], s1, preferred_element_type=jnp.float32) / cnt
    s2g = jnp.dot(gmat_ref[...], s2, preferred_element_type=jnp.float32) / cnt
    var = s2g - s1g * s1g
    scale = jax.lax.rsqrt(var + _EPS) * g_ref[...]
    shift = b_ref[...] - s1g * scale

    for i in range(d):
        y = acc_ref[:, i * hw2:(i + 1) * hw2] * scale + shift
        y = jnp.maximum(y, 0.0).astype(jnp.bfloat16)
        o_ref[0, :, (2 * i) * hw2:(2 * i + 1) * hw2] = y[:cout]
        o_ref[0, :, (2 * i + 1) * hw2:(2 * i + 2) * hw2] = y[cout:]


def _axis_up_matrix(L):
    i = jnp.arange(L)
    m = jnp.zeros((2 * L, L), jnp.float32)
    m = m.at[2 * i, jnp.clip(i - 1, 0, L - 1)].add(0.25)
    m = m.at[2 * i, i].add(0.75)
    m = m.at[2 * i + 1, i].add(0.75)
    m = m.at[2 * i + 1, jnp.clip(i + 1, 0, L - 1)].add(0.25)
    return m


def _depth_fold_coeffs():
    c = [[[[0.0] * 3 for _ in range(3)] for _ in range(2)] for _ in range(3)]
    c[0][0][0] = [0.75, 0.25, 0.0]
    c[0][0][1] = [0.25, 0.75, 0.75]
    c[0][0][2] = [0.0, 0.0, 0.25]
    c[0][1][0] = [0.25, 0.0, 0.0]
    c[0][1][1] = [0.75, 0.75, 0.25]
    c[0][1][2] = [0.0, 0.25, 0.75]
    c[1][0][0] = [0.0, 1.0, 0.75]
    c[1][0][1] = [0.0, 0.0, 0.25]
    c[1][1][0] = [1.0, 0.75, 0.25]
    c[1][1][1] = [0.0, 0.25, 0.75]
    c[2][0][1] = [0.75, 0.25, 0.0]
    c[2][0][2] = [0.25, 0.75, 1.0]
    c[2][1][1] = [0.25, 0.0, 0.0]
    c[2][1][2] = [0.75, 1.0, 0.0]
    return jnp.asarray(c, jnp.float32)


@jax.jit
def _forward(w, gamma, beta, x):
    n, cin, d, h, wd = x.shape
    cout = w.shape[0]
    d2, h2, w2 = 2 * d, 2 * h, 2 * wd
    hw, hw2, r2 = h * wd, h2 * w2, d2 * h2 * w2
    c2 = 2 * cout

    kt = jnp.kron(_axis_up_matrix(h), _axis_up_matrix(wd)).T
    kt3 = kt.reshape(hw, h2, w2)
    ktp = jnp.pad(kt3, ((0, 0), (1, 1), (1, 1)))
    blocks = [ktp[:, 1 + dh:1 + dh + h2, 1 + dw:1 + dw + w2].reshape(hw, hw2)
              for dh in (-1, 0, 1) for dw in (-1, 0, 1)]
    kt9 = jnp.concatenate(blocks, axis=1).astype(jnp.bfloat16)

    coef = _depth_fold_coeffs()
    wp = jnp.transpose(w, (2, 3, 4, 0, 1))
    a = jnp.einsum('vptk,kabyc->vtpyabc', coef, wp)
    a2 = a.reshape(3, 3, c2, 9 * cin).astype(jnp.bfloat16)

    ch = jnp.arange(c2) % cout
    gid = ch // (cout // _GROUPS)
    gmat = (gid[:, None] == gid[None, :]).astype(jnp.float32)

    g2 = jnp.concatenate([gamma, gamma]).reshape(c2, 1).astype(jnp.float32)
    b2 = jnp.concatenate([beta, beta]).reshape(c2, 1).astype(jnp.float32)

    x_r = x.reshape(n, cin, d * hw)

    body = functools.partial(_fused_kernel, d=d, cin=cin, cout=cout,
                             hw=hw, hw2=hw2, groups=_GROUPS)
    out = pl.pallas_call(
        body,
        out_shape=jax.ShapeDtypeStruct((n, cout, r2), jnp.bfloat16),
        grid_spec=pltpu.PrefetchScalarGridSpec(
            num_scalar_prefetch=0,
            grid=(n,),
            in_specs=[
                pl.BlockSpec((1, cin, d * hw), lambda i: (i, 0, 0)),
                pl.BlockSpec((hw, 9 * hw2), lambda i: (0, 0)),
                pl.BlockSpec((3, 3, c2, 9 * cin), lambda i: (0, 0, 0, 0)),
                pl.BlockSpec((c2, c2), lambda i: (0, 0)),
                pl.BlockSpec((c2, 1), lambda i: (0, 0)),
                pl.BlockSpec((c2, 1), lambda i: (0, 0)),
            ],
            out_specs=pl.BlockSpec((1, cout, r2), lambda i: (i, 0, 0)),
            scratch_shapes=[
                pltpu.VMEM((d * cin, hw), jnp.bfloat16),
                pltpu.VMEM((d * cin, 9 * hw2), jnp.bfloat16),
                pltpu.VMEM((9 * cin, d * hw2), jnp.bfloat16),
                pltpu.VMEM((c2, d * hw2), jnp.float32),
            ],
        ),
        compiler_params=pltpu.CompilerParams(
            dimension_semantics=("parallel",)),
    )(x_r, kt9, a2, gmat, g2, b2)

    return out.reshape(n, cout, d2, h2, w2).astype(jnp.float32)


def kernel(w, gamma, beta, x):
    return _forward(w, gamma, beta, x)


# speedup vs baseline: 2.4849x; 1.0026x over previous
import functools

import jax
import jax.numpy as jnp
from jax.experimental import pallas as pl
from jax.experimental.pallas import tpu as pltpu

_EPS = 1e-5
_GROUPS = 8


def _fused_kernel(x_ref, kt9_ref, a2_ref, gmat_ref, g_ref, b_ref, o_ref,
                  xt_ref, xu9_ref, z_ref, acc_ref, *, d, cin, cout, hw, hw2,
                  groups):
    c2 = 2 * cout

    for i in range(d):
        xt_ref[i * cin:(i + 1) * cin, :] = (
            x_ref[0, :, i * hw:(i + 1) * hw].astype(jnp.bfloat16))

    xu9_ref[...] = jnp.dot(xt_ref[...], kt9_ref[...],
                           preferred_element_type=jnp.float32
                           ).astype(jnp.bfloat16)

    for i in range(d):
        for k in range(9):
            z_ref[k * cin:(k + 1) * cin, i * hw2:(i + 1) * hw2] = \
                xu9_ref[i * cin:(i + 1) * cin, k * hw2:(k + 1) * hw2]

    def conv_group(v, ls, lo, wlen):
        acc = jnp.zeros((c2, wlen * hw2), jnp.float32)
        for t in range(3):
            acc = acc + jnp.dot(
                a2_ref[v, t],
                z_ref[:, (ls + t) * hw2:(ls + t + wlen) * hw2],
                preferred_element_type=jnp.float32)
        acc_ref[:, lo * hw2:(lo + wlen) * hw2] = acc

    conv_group(1, 0, 0, 1)
    i = 1
    while i < d - 1:
        wlen = min(2, d - 1 - i)
        conv_group(0, i - 1, i, wlen)
        i += wlen
    conv_group(2, d - 3, d - 1, 1)

    s1 = jnp.zeros((c2, 1), jnp.float32)
    s2 = jnp.zeros((c2, 1), jnp.float32)
    for i in range(d):
        blk = acc_ref[:, i * hw2:(i + 1) * hw2]
        s1 = s1 + jnp.sum(blk, axis=1, keepdims=True)
        s2 = s2 + jnp.sum(blk * blk, axis=1, keepdims=True)
    cnt = float((cout // groups) * 2 * d * hw2)
    s1g = jnp.dot(gmat_ref[...], s1, preferred_element_type=jnp.float32) / cnt
    s2g = jnp.dot(gmat_ref[...], s2, preferred_element_type=jnp.float32) / cnt
    var = s2g - s1g * s1g
    scale = jax.lax.rsqrt(var + _EPS) * g_ref[...]
    shift = b_ref[...] - s1g * scale

    for i in range(d):
        y = acc_ref[:, i * hw2:(i + 1) * hw2] * scale + shift
        y = jnp.maximum(y, 0.0).astype(jnp.bfloat16)
        o_ref[0, :, (2 * i) * hw2:(2 * i + 1) * hw2] = y[:cout]
        o_ref[0, :, (2 * i + 1) * hw2:(2 * i + 2) * hw2] = y[cout:]


def _axis_up_matrix(L):
    i = jnp.arange(L)
    m = jnp.zeros((2 * L, L), jnp.float32)
    m = m.at[2 * i, jnp.clip(i - 1, 0, L - 1)].add(0.25)
    m = m.at[2 * i, i].add(0.75)
    m = m.at[2 * i + 1, i].add(0.75)
    m = m.at[2 * i + 1, jnp.clip(i + 1, 0, L - 1)].add(0.25)
    return m


def _depth_fold_coeffs():
    c = [[[[0.0] * 3 for _ in range(3)] for _ in range(2)] for _ in range(3)]
    c[0][0][0] = [0.75, 0.25, 0.0]
    c[0][0][1] = [0.25, 0.75, 0.75]
    c[0][0][2] = [0.0, 0.0, 0.25]
    c[0][1][0] = [0.25, 0.0, 0.0]
    c[0][1][1] = [0.75, 0.75, 0.25]
    c[0][1][2] = [0.0, 0.25, 0.75]
    c[1][0][0] = [0.0, 1.0, 0.75]
    c[1][0][1] = [0.0, 0.0, 0.25]
    c[1][1][0] = [1.0, 0.75, 0.25]
    c[1][1][1] = [0.0, 0.25, 0.75]
    c[2][0][1] = [0.75, 0.25, 0.0]
    c[2][0][2] = [0.25, 0.75, 1.0]
    c[2][1][1] = [0.25, 0.0, 0.0]
    c[2][1][2] = [0.75, 1.0, 0.0]
    return jnp.asarray(c, jnp.float32)


@jax.jit
def _forward(w, gamma, beta, x):
    n, cin, d, h, wd = x.shape
    cout = w.shape[0]
    d2, h2, w2 = 2 * d, 2 * h, 2 * wd
    hw, hw2, r2 = h * wd, h2 * w2, d2 * h2 * w2
    c2 = 2 * cout

    kt = jnp.kron(_axis_up_matrix(h), _axis_up_matrix(wd)).T
    kt3 = kt.reshape(hw, h2, w2)
    ktp = jnp.pad(kt3, ((0, 0), (1, 1), (1, 1)))
    blocks = [ktp[:, 1 + dh:1 + dh + h2, 1 + dw:1 + dw + w2].reshape(hw, hw2)
              for dh in (-1, 0, 1) for dw in (-1, 0, 1)]
    kt9 = jnp.concatenate(blocks, axis=1).astype(jnp.bfloat16)

    coef = _depth_fold_coeffs()
    wp = jnp.transpose(w, (2, 3, 4, 0, 1))
    a = jnp.einsum('vptk,kabyc->vtpyabc', coef, wp)
    a2 = a.reshape(3, 3, c2, 9 * cin).astype(jnp.bfloat16)

    ch = jnp.arange(c2) % cout
    gid = ch // (cout // _GROUPS)
    gmat = (gid[:, None] == gid[None, :]).astype(jnp.float32)

    g2 = jnp.concatenate([gamma, gamma]).reshape(c2, 1).astype(jnp.float32)
    b2 = jnp.concatenate([beta, beta]).reshape(c2, 1).astype(jnp.float32)

    x_r = x.reshape(n, cin, d * hw)

    body = functools.partial(_fused_kernel, d=d, cin=cin, cout=cout,
                             hw=hw, hw2=hw2, groups=_GROUPS)
    out = pl.pallas_call(
        body,
        out_shape=jax.ShapeDtypeStruct((n, cout, r2), jnp.bfloat16),
        grid_spec=pltpu.PrefetchScalarGridSpec(
            num_scalar_prefetch=0,
            grid=(n,),
            in_specs=[
                pl.BlockSpec((1, cin, d * hw), lambda i: (i, 0, 0)),
                pl.BlockSpec((hw, 9 * hw2), lambda i: (0, 0)),
                pl.BlockSpec((3, 3, c2, 9 * cin), lambda i: (0, 0, 0, 0)),
                pl.BlockSpec((c2, c2), lambda i: (0, 0)),
                pl.BlockSpec((c2, 1), lambda i: (0, 0)),
                pl.BlockSpec((c2, 1), lambda i: (0, 0)),
            ],
            out_specs=pl.BlockSpec((1, cout, r2), lambda i: (i, 0, 0)),
            scratch_shapes=[
                pltpu.VMEM((d * cin, hw), jnp.bfloat16),
                pltpu.VMEM((d * cin, 9 * hw2), jnp.bfloat16),
                pltpu.VMEM((9 * cin, d * hw2), jnp.bfloat16),
                pltpu.VMEM((c2, d * hw2), jnp.float32),
            ],
        ),
        compiler_params=pltpu.CompilerParams(
            dimension_semantics=("parallel",)),
    )(x_r, kt9, a2, gmat, g2, b2)

    return out.reshape(n, cout, d2, h2, w2).astype(jnp.float32)


def kernel(w, gamma, beta, x):
    return _forward(w, gamma, beta, x)


# speedup vs baseline: 2.6271x; 1.0572x over previous
import functools

import jax
import jax.numpy as jnp
from jax.experimental import pallas as pl
from jax.experimental.pallas import tpu as pltpu

_EPS = 1e-5
_GROUPS = 8


def _fused_kernel(x_ref, kt9_ref, a2_ref, gmat_ref, g_ref, b_ref, o_ref,
                  xt_ref, xu9_ref, z_ref, acc_ref, *, nb, d, cin, cout, hw,
                  hw2, groups):
    c2 = 2 * cout

    for s in range(nb):
        for i in range(d):
            xt_ref[i * cin:(i + 1) * cin, :] = (
                x_ref[s, :, i * hw:(i + 1) * hw].astype(jnp.bfloat16))

        xu9_ref[...] = jnp.dot(xt_ref[...], kt9_ref[...],
                               preferred_element_type=jnp.float32
                               ).astype(jnp.bfloat16)

        for i in range(d):
            for k in range(9):
                z_ref[k * cin:(k + 1) * cin, i * hw2:(i + 1) * hw2] = \
                    xu9_ref[i * cin:(i + 1) * cin, k * hw2:(k + 1) * hw2]

        def conv_group(v, ls, lo, wlen):
            acc = jnp.zeros((c2, wlen * hw2), jnp.float32)
            for t in range(3):
                acc = acc + jnp.dot(
                    a2_ref[v, t],
                    z_ref[:, (ls + t) * hw2:(ls + t + wlen) * hw2],
                    preferred_element_type=jnp.float32)
            acc_ref[:, lo * hw2:(lo + wlen) * hw2] = acc

        conv_group(1, 0, 0, 1)
        i = 1
        while i < d - 1:
            wlen = min(2, d - 1 - i)
            conv_group(0, i - 1, i, wlen)
            i += wlen
        conv_group(2, d - 3, d - 1, 1)

        s1 = jnp.zeros((c2, 1), jnp.float32)
        s2 = jnp.zeros((c2, 1), jnp.float32)
        for i in range(d):
            blk = acc_ref[:, i * hw2:(i + 1) * hw2]
            s1 = s1 + jnp.sum(blk, axis=1, keepdims=True)
            s2 = s2 + jnp.sum(blk * blk, axis=1, keepdims=True)
        cnt = float((cout // groups) * 2 * d * hw2)
        s1g = jnp.dot(gmat_ref[...], s1,
                      preferred_element_type=jnp.float32) / cnt
        s2g = jnp.dot(gmat_ref[...], s2,
                      preferred_element_type=jnp.float32) / cnt
        var = s2g - s1g * s1g
        scale = jax.lax.rsqrt(var + _EPS) * g_ref[...]
        shift = b_ref[...] - s1g * scale

        for i in range(d):
            y = acc_ref[:, i * hw2:(i + 1) * hw2] * scale + shift
            y = jnp.maximum(y, 0.0).astype(jnp.bfloat16)
            o_ref[s, :, (2 * i) * hw2:(2 * i + 1) * hw2] = y[:cout]
            o_ref[s, :, (2 * i + 1) * hw2:(2 * i + 2) * hw2] = y[cout:]


def _axis_up_matrix(L):
    jj = jnp.arange(2 * L)[:, None]
    ii = jnp.arange(L)[None, :]
    i0 = jj // 2
    near = 0.75 * (ii == i0)
    ev = near + 0.25 * (ii == jnp.clip(i0 - 1, 0, L - 1))
    od = near + 0.25 * (ii == jnp.clip(i0 + 1, 0, L - 1))
    return jnp.where(jj % 2 == 0, ev, od).astype(jnp.float32)


def _depth_fold_coeffs():
    c = [[[[0.0] * 3 for _ in range(3)] for _ in range(2)] for _ in range(3)]
    c[0][0][0] = [0.75, 0.25, 0.0]
    c[0][0][1] = [0.25, 0.75, 0.75]
    c[0][0][2] = [0.0, 0.0, 0.25]
    c[0][1][0] = [0.25, 0.0, 0.0]
    c[0][1][1] = [0.75, 0.75, 0.25]
    c[0][1][2] = [0.0, 0.25, 0.75]
    c[1][0][0] = [0.0, 1.0, 0.75]
    c[1][0][1] = [0.0, 0.0, 0.25]
    c[1][1][0] = [1.0, 0.75, 0.25]
    c[1][1][1] = [0.0, 0.25, 0.75]
    c[2][0][1] = [0.75, 0.25, 0.0]
    c[2][0][2] = [0.25, 0.75, 1.0]
    c[2][1][1] = [0.25, 0.0, 0.0]
    c[2][1][2] = [0.75, 1.0, 0.0]
    return jnp.asarray(c, jnp.float32)


@jax.jit
def _forward(w, gamma, beta, x):
    n, cin, d, h, wd = x.shape
    cout = w.shape[0]
    d2, h2, w2 = 2 * d, 2 * h, 2 * wd
    hw, hw2, r2 = h * wd, h2 * w2, d2 * h2 * w2
    c2 = 2 * cout

    kt = jnp.kron(_axis_up_matrix(h), _axis_up_matrix(wd)).T
    kt3 = kt.reshape(hw, h2, w2)
    ktp = jnp.pad(kt3, ((0, 0), (1, 1), (1, 1)))
    blocks = [ktp[:, 1 + dh:1 + dh + h2, 1 + dw:1 + dw + w2].reshape(hw, hw2)
              for dh in (-1, 0, 1) for dw in (-1, 0, 1)]
    kt9 = jnp.concatenate(blocks, axis=1).astype(jnp.bfloat16)

    coef = _depth_fold_coeffs()
    wp = jnp.transpose(w, (2, 3, 4, 0, 1))
    a = jnp.einsum('vptk,kabyc->vtpyabc', coef, wp)
    a2 = a.reshape(3, 3, c2, 9 * cin).astype(jnp.bfloat16)

    ch = jnp.arange(c2) % cout
    gid = ch // (cout // _GROUPS)
    gmat = (gid[:, None] == gid[None, :]).astype(jnp.float32)

    g2 = jnp.concatenate([gamma, gamma]).reshape(c2, 1).astype(jnp.float32)
    b2 = jnp.concatenate([beta, beta]).reshape(c2, 1).astype(jnp.float32)

    x_r = x.reshape(n, cin, d * hw)

    nb = 2 if n % 2 == 0 else 1
    body = functools.partial(_fused_kernel, nb=nb, d=d, cin=cin, cout=cout,
                             hw=hw, hw2=hw2, groups=_GROUPS)
    out = pl.pallas_call(
        body,
        out_shape=jax.ShapeDtypeStruct((n, cout, r2), jnp.bfloat16),
        grid_spec=pltpu.PrefetchScalarGridSpec(
            num_scalar_prefetch=0,
            grid=(n // nb,),
            in_specs=[
                pl.BlockSpec((nb, cin, d * hw), lambda i: (i, 0, 0)),
                pl.BlockSpec((hw, 9 * hw2), lambda i: (0, 0)),
                pl.BlockSpec((3, 3, c2, 9 * cin), lambda i: (0, 0, 0, 0)),
                pl.BlockSpec((c2, c2), lambda i: (0, 0)),
                pl.BlockSpec((c2, 1), lambda i: (0, 0)),
                pl.BlockSpec((c2, 1), lambda i: (0, 0)),
            ],
            out_specs=pl.BlockSpec((nb, cout, r2), lambda i: (i, 0, 0)),
            scratch_shapes=[
                pltpu.VMEM((d * cin, hw), jnp.bfloat16),
                pltpu.VMEM((d * cin, 9 * hw2), jnp.bfloat16),
                pltpu.VMEM((9 * cin, d * hw2), jnp.bfloat16),
                pltpu.VMEM((c2, d * hw2), jnp.float32),
            ],
        ),
        compiler_params=pltpu.CompilerParams(
            dimension_semantics=("parallel",)),
    )(x_r, kt9, a2, gmat, g2, b2)

    return out.reshape(n, cout, d2, h2, w2).astype(jnp.float32)


def kernel(w, gamma, beta, x):
    return _forward(w, gamma, beta, x)


# speedup vs baseline: 2.6298x; 1.0010x over previous
import functools

import jax
import jax.numpy as jnp
from jax.experimental import pallas as pl
from jax.experimental.pallas import tpu as pltpu

_EPS = 1e-5
_GROUPS = 8


def _fused_kernel(x_ref, kt9_ref, a2_ref, gmat_ref, g_ref, b_ref, o_ref,
                  xt_ref, xu9_ref, z_ref, acc_ref, *, nb, d, cin, cout, hw,
                  hw2, groups):
    c2 = 2 * cout

    for s in range(nb):
        for i in range(d):
            xt_ref[i * cin:(i + 1) * cin, :] = (
                x_ref[s, :, i * hw:(i + 1) * hw].astype(jnp.bfloat16))

        xu9_ref[...] = jnp.dot(xt_ref[...], kt9_ref[...],
                               preferred_element_type=jnp.float32
                               ).astype(jnp.bfloat16)

        for i in range(d):
            for k in range(9):
                z_ref[k * cin:(k + 1) * cin, i * hw2:(i + 1) * hw2] = \
                    xu9_ref[i * cin:(i + 1) * cin, k * hw2:(k + 1) * hw2]

        def conv_group(v, ls, lo, wlen):
            acc = jnp.zeros((c2, wlen * hw2), jnp.float32)
            for t in range(3):
                acc = acc + jnp.dot(
                    a2_ref[v, t],
                    z_ref[:, (ls + t) * hw2:(ls + t + wlen) * hw2],
                    preferred_element_type=jnp.float32)
            acc_ref[:, lo * hw2:(lo + wlen) * hw2] = acc

        conv_group(1, 0, 0, 1)
        i = 1
        while i < d - 1:
            wlen = min(4, d - 1 - i)
            conv_group(0, i - 1, i, wlen)
            i += wlen
        conv_group(2, d - 3, d - 1, 1)

        s1 = jnp.zeros((c2, 1), jnp.float32)
        s2 = jnp.zeros((c2, 1), jnp.float32)
        for i in range(d):
            blk = acc_ref[:, i * hw2:(i + 1) * hw2]
            s1 = s1 + jnp.sum(blk, axis=1, keepdims=True)
            s2 = s2 + jnp.sum(blk * blk, axis=1, keepdims=True)
        cnt = float((cout // groups) * 2 * d * hw2)
        s1g = jnp.dot(gmat_ref[...], s1,
                      preferred_element_type=jnp.float32) / cnt
        s2g = jnp.dot(gmat_ref[...], s2,
                      preferred_element_type=jnp.float32) / cnt
        var = s2g - s1g * s1g
        scale = jax.lax.rsqrt(var + _EPS) * g_ref[...]
        shift = b_ref[...] - s1g * scale

        for i in range(d):
            y = acc_ref[:, i * hw2:(i + 1) * hw2] * scale + shift
            y = jnp.maximum(y, 0.0).astype(jnp.bfloat16)
            o_ref[s, :, (2 * i) * hw2:(2 * i + 1) * hw2] = y[:cout]
            o_ref[s, :, (2 * i + 1) * hw2:(2 * i + 2) * hw2] = y[cout:]


def _axis_up_matrix(L):
    jj = jnp.arange(2 * L)[:, None]
    ii = jnp.arange(L)[None, :]
    i0 = jj // 2
    near = 0.75 * (ii == i0)
    ev = near + 0.25 * (ii == jnp.clip(i0 - 1, 0, L - 1))
    od = near + 0.25 * (ii == jnp.clip(i0 + 1, 0, L - 1))
    return jnp.where(jj % 2 == 0, ev, od).astype(jnp.float32)


def _depth_fold_coeffs():
    c = [[[[0.0] * 3 for _ in range(3)] for _ in range(2)] for _ in range(3)]
    c[0][0][0] = [0.75, 0.25, 0.0]
    c[0][0][1] = [0.25, 0.75, 0.75]
    c[0][0][2] = [0.0, 0.0, 0.25]
    c[0][1][0] = [0.25, 0.0, 0.0]
    c[0][1][1] = [0.75, 0.75, 0.25]
    c[0][1][2] = [0.0, 0.25, 0.75]
    c[1][0][0] = [0.0, 1.0, 0.75]
    c[1][0][1] = [0.0, 0.0, 0.25]
    c[1][1][0] = [1.0, 0.75, 0.25]
    c[1][1][1] = [0.0, 0.25, 0.75]
    c[2][0][1] = [0.75, 0.25, 0.0]
    c[2][0][2] = [0.25, 0.75, 1.0]
    c[2][1][1] = [0.25, 0.0, 0.0]
    c[2][1][2] = [0.75, 1.0, 0.0]
    return jnp.asarray(c, jnp.float32)


@jax.jit
def _forward(w, gamma, beta, x):
    n, cin, d, h, wd = x.shape
    cout = w.shape[0]
    d2, h2, w2 = 2 * d, 2 * h, 2 * wd
    hw, hw2, r2 = h * wd, h2 * w2, d2 * h2 * w2
    c2 = 2 * cout

    kt = jnp.kron(_axis_up_matrix(h), _axis_up_matrix(wd)).T
    kt3 = kt.reshape(hw, h2, w2)
    ktp = jnp.pad(kt3, ((0, 0), (1, 1), (1, 1)))
    blocks = [ktp[:, 1 + dh:1 + dh + h2, 1 + dw:1 + dw + w2].reshape(hw, hw2)
              for dh in (-1, 0, 1) for dw in (-1, 0, 1)]
    kt9 = jnp.concatenate(blocks, axis=1).astype(jnp.bfloat16)

    coef = _depth_fold_coeffs()
    wp = jnp.transpose(w, (2, 3, 4, 0, 1))
    a = jnp.einsum('vptk,kabyc->vtpyabc', coef, wp)
    a2 = a.reshape(3, 3, c2, 9 * cin).astype(jnp.bfloat16)

    ch = jnp.arange(c2) % cout
    gid = ch // (cout // _GROUPS)
    gmat = (gid[:, None] == gid[None, :]).astype(jnp.float32)

    g2 = jnp.concatenate([gamma, gamma]).reshape(c2, 1).astype(jnp.float32)
    b2 = jnp.concatenate([beta, beta]).reshape(c2, 1).astype(jnp.float32)

    x_r = x.reshape(n, cin, d * hw)

    nb = 2 if n % 2 == 0 else 1
    body = functools.partial(_fused_kernel, nb=nb, d=d, cin=cin, cout=cout,
                             hw=hw, hw2=hw2, groups=_GROUPS)
    out = pl.pallas_call(
        body,
        out_shape=jax.ShapeDtypeStruct((n, cout, r2), jnp.bfloat16),
        grid_spec=pltpu.PrefetchScalarGridSpec(
            num_scalar_prefetch=0,
            grid=(n // nb,),
            in_specs=[
                pl.BlockSpec((nb, cin, d * hw), lambda i: (i, 0, 0)),
                pl.BlockSpec((hw, 9 * hw2), lambda i: (0, 0)),
                pl.BlockSpec((3, 3, c2, 9 * cin), lambda i: (0, 0, 0, 0)),
                pl.BlockSpec((c2, c2), lambda i: (0, 0)),
                pl.BlockSpec((c2, 1), lambda i: (0, 0)),
                pl.BlockSpec((c2, 1), lambda i: (0, 0)),
            ],
            out_specs=pl.BlockSpec((nb, cout, r2), lambda i: (i, 0, 0)),
            scratch_shapes=[
                pltpu.VMEM((d * cin, hw), jnp.bfloat16),
                pltpu.VMEM((d * cin, 9 * hw2), jnp.bfloat16),
                pltpu.VMEM((9 * cin, d * hw2), jnp.bfloat16),
                pltpu.VMEM((c2, d * hw2), jnp.float32),
            ],
        ),
        compiler_params=pltpu.CompilerParams(
            dimension_semantics=("parallel",)),
    )(x_r, kt9, a2, gmat, g2, b2)

    return out.reshape(n, cout, d2, h2, w2).astype(jnp.float32)


def kernel(w, gamma, beta, x):
    return _forward(w, gamma, beta, x)
